# Initial kernel scaffold; baseline (speedup 1.0000x reference)
import jax
import jax.numpy as jnp
from jax.experimental import pallas as pl


def kernel(x, table):
    raise NotImplementedError("write your pallas kernel here")


# speedup vs baseline: 3.6190x; 3.6190x over previous
import functools

import jax
import jax.numpy as jnp
from jax import lax
from jax.experimental import pallas as pl
from jax.experimental.pallas import tpu as pltpu
from jax.experimental.pallas import tpu_sc as plsc


def _make_sc_broadcast(batch, seq_len, d_model, dtype):
    info = plsc.get_sparse_core_info()
    num_workers = info.num_cores * info.num_subcores
    rows_per_worker = seq_len // num_workers
    chunk = min(64, rows_per_worker)
    num_chunks = rows_per_worker // chunk

    mesh = plsc.VectorSubcoreMesh(core_axis_name="c", subcore_axis_name="s")

    @functools.partial(
        pl.kernel,
        mesh=mesh,
        out_type=jax.ShapeDtypeStruct((batch, seq_len, d_model), dtype),
        scratch_types=[
            pltpu.VMEM((chunk, d_model), dtype),
            pltpu.SemaphoreType.DMA,
        ],
    )
    def sc_broadcast(table_hbm, out_hbm, buf, sem):
        wid = lax.axis_index("s") * info.num_cores + lax.axis_index("c")
        base = wid * rows_per_worker

        def body(i, carry):
            r0 = base + i * chunk
            pltpu.sync_copy(table_hbm.at[pl.ds(r0, chunk)], buf)
            for b in range(batch):
                pltpu.sync_copy(buf, out_hbm.at[b, pl.ds(r0, chunk)])
            return carry

        lax.fori_loop(0, num_chunks, body, 0)

    return sc_broadcast


def kernel(x, table):
    batch, seq_len, d_model = x.shape
    fn = _make_sc_broadcast(batch, seq_len, d_model, table.dtype)
    return fn(table)
